# baseline (speedup 1.0000x reference)
import jax
import jax.numpy as jnp
from jax.experimental import pallas as pl
from jax.experimental.pallas import tpu as pltpu

_R = 64
_C = 1_000_000
_SUB = 32
_CW = _C // _SUB


def _fused_kernel(d_ref, m_ref, o_ref, idx_ref):
    r = pl.program_id(0)
    x = d_ref[0]
    msk = m_ref[0]

    mx = jnp.max(jnp.max(x, axis=1, keepdims=True), axis=0, keepdims=True)
    e = jnp.exp(x - mx)
    s = jnp.sum(jnp.sum(e, axis=1, keepdims=True), axis=0, keepdims=True)
    p = e * (1.0 / s)
    o_ref[0] = p

    q = jnp.where(msk, jnp.float32(0.0), p + jnp.float32(1e-5))
    qmax = jnp.max(jnp.max(q, axis=1, keepdims=True), axis=0, keepdims=True)
    sub_i = jax.lax.broadcasted_iota(jnp.int32, (_SUB, _CW), 0)
    lane_i = jax.lax.broadcasted_iota(jnp.int32, (_SUB, _CW), 1)
    gidx = sub_i * _CW + lane_i
    cand = jnp.where(q == qmax, gidx, jnp.int32(_C))
    best = jnp.min(jnp.min(cand, axis=1, keepdims=True), axis=0, keepdims=True)
    idx_ref[pl.ds(r, 1), :] = best


def kernel(dist, mask):
    d3 = dist.reshape(_R, _SUB, _CW)
    m3 = mask.reshape(_R, _SUB, _CW)
    out3, idx = pl.pallas_call(
        _fused_kernel,
        grid=(_R,),
        in_specs=[
            pl.BlockSpec((1, _SUB, _CW), lambda r: (r, 0, 0)),
            pl.BlockSpec((1, _SUB, _CW), lambda r: (r, 0, 0)),
        ],
        out_specs=[
            pl.BlockSpec((1, _SUB, _CW), lambda r: (r, 0, 0)),
            pl.BlockSpec((_R, 1), lambda r: (0, 0)),
        ],
        out_shape=[
            jax.ShapeDtypeStruct((_R, _SUB, _CW), jnp.float32),
            jax.ShapeDtypeStruct((_R, 1), jnp.int32),
        ],
        compiler_params=pltpu.CompilerParams(
            dimension_semantics=("arbitrary",),
        ),
    )(d3, m3)
    return out3.reshape(_R, _C), idx


# speedup vs baseline: 3.2966x; 3.2966x over previous
import jax
import jax.numpy as jnp
from jax.experimental import pallas as pl
from jax.experimental.pallas import tpu as pltpu

_R = 64
_C = 1_000_000
_BC = 8192
_CB = (_C + _BC - 1) // _BC

_NEG_INF = float("-inf")


def _pass1_kernel(d_ref, m_out, s_out, m_s, s_s):
    j = pl.program_id(0)
    x = d_ref[...]
    col = jax.lax.broadcasted_iota(jnp.int32, (_R, _BC), 1) + j * _BC
    x = jnp.where(col < _C, x, jnp.float32(_NEG_INF))
    bm = jnp.max(x, axis=1, keepdims=True)

    @pl.when(j == 0)
    def _init():
        m_s[...] = bm
        s_s[...] = jnp.sum(jnp.exp(x - bm), axis=1, keepdims=True)

    @pl.when(j > 0)
    def _acc():
        m_prev = m_s[...]
        m_new = jnp.maximum(m_prev, bm)
        s_s[...] = (s_s[...] * jnp.exp(m_prev - m_new)
                    + jnp.sum(jnp.exp(x - m_new), axis=1, keepdims=True))
        m_s[...] = m_new

    @pl.when(j == _CB - 1)
    def _fin():
        m_out[...] = m_s[...]
        s_out[...] = s_s[...]


def _pass2_kernel(d_ref, k_ref, m_ref, s_ref, o_ref, idx_ref, gval_s, gidx_s):
    j = pl.program_id(0)
    x = d_ref[...]
    msk = k_ref[...]
    m = m_ref[...]
    inv_s = 1.0 / s_ref[...]

    p = jnp.exp(x - m) * inv_s
    o_ref[...] = p

    col = jax.lax.broadcasted_iota(jnp.int32, (_R, _BC), 1) + j * _BC
    valid = jnp.logical_and(col < _C, jnp.logical_not(msk))
    q = jnp.where(valid, p + jnp.float32(1e-5), jnp.float32(0.0))
    bq = jnp.max(q, axis=1, keepdims=True)
    cand = jnp.where(q == bq, col, jnp.int32(_C))
    bi = jnp.min(cand, axis=1, keepdims=True)

    @pl.when(j == 0)
    def _init():
        gval_s[...] = jnp.full((_R, 1), -1.0, jnp.float32)
        gidx_s[...] = jnp.zeros((_R, 1), jnp.int32)

    upd = bq > gval_s[...]
    gval_s[...] = jnp.where(upd, bq, gval_s[...])
    gidx_s[...] = jnp.where(upd, bi, gidx_s[...])

    @pl.when(j == _CB - 1)
    def _fin():
        idx_ref[...] = gidx_s[...]


def kernel(dist, mask):
    m, s = pl.pallas_call(
        _pass1_kernel,
        grid=(_CB,),
        in_specs=[pl.BlockSpec((_R, _BC), lambda j: (0, j))],
        out_specs=[
            pl.BlockSpec((_R, 1), lambda j: (0, 0)),
            pl.BlockSpec((_R, 1), lambda j: (0, 0)),
        ],
        out_shape=[
            jax.ShapeDtypeStruct((_R, 1), jnp.float32),
            jax.ShapeDtypeStruct((_R, 1), jnp.float32),
        ],
        scratch_shapes=[
            pltpu.VMEM((_R, 1), jnp.float32),
            pltpu.VMEM((_R, 1), jnp.float32),
        ],
        compiler_params=pltpu.CompilerParams(
            dimension_semantics=("arbitrary",),
        ),
    )(dist)

    out, idx = pl.pallas_call(
        _pass2_kernel,
        grid=(_CB,),
        in_specs=[
            pl.BlockSpec((_R, _BC), lambda j: (0, j)),
            pl.BlockSpec((_R, _BC), lambda j: (0, j)),
            pl.BlockSpec((_R, 1), lambda j: (0, 0)),
            pl.BlockSpec((_R, 1), lambda j: (0, 0)),
        ],
        out_specs=[
            pl.BlockSpec((_R, _BC), lambda j: (0, j)),
            pl.BlockSpec((_R, 1), lambda j: (0, 0)),
        ],
        out_shape=[
            jax.ShapeDtypeStruct((_R, _C), jnp.float32),
            jax.ShapeDtypeStruct((_R, 1), jnp.int32),
        ],
        scratch_shapes=[
            pltpu.VMEM((_R, 1), jnp.float32),
            pltpu.VMEM((_R, 1), jnp.int32),
        ],
        compiler_params=pltpu.CompilerParams(
            dimension_semantics=("arbitrary",),
        ),
    )(dist, mask, m, s)
    return out, idx
